# baseline (speedup 1.0000x reference)
import jax
import jax.numpy as jnp
from jax import lax
from jax.experimental import pallas as pl
from jax.experimental.pallas import tpu as pltpu
from jax.experimental.pallas import tpu_sc as plsc

N = 10000
E = 160000
C = 256
N_LAYERS = 3
ALPHA = 0.9

NC = 2
NS = 16
L = 16
CH = C // NC
NP = 10240
K = 64
RPT = NP // NS
R = 32
NRB = RPT // R
ZR = 16
NCH = CH // L


def _lpa_body(y2, maskf, srcs_h, dsts_h, ws_h, bnds_h,
              out2, bufA, bufB, res_h,
              acc, src2, dst2, w2, bnds_v,
              rowbuf2, wbuf, rbuf, zbuf, mbuf, bexp,
              gsem0, gsem1, ssem0, ssem1, tsem0, tsem1):
    c = lax.axis_index("c")
    s = lax.axis_index("s")
    row0 = s * RPT
    gsems = (gsem0, gsem1)
    ssems = (ssem0, ssem1)
    tsems = (tsem0, tsem1)

    pltpu.sync_copy(bnds_h.at[s], bnds_v)
    bv = bnds_v[...]
    start = bv[0]
    end = bv[1]
    b0 = start // K
    b1 = jnp.where(end > start, (end - 1) // K + 1, b0)
    nb = b1 - b0

    def zbody(i, _):
        for cc in range(NCH):
            zbuf[i, pl.ds(cc * L, L)] = jnp.zeros((L,), jnp.float32)
        return 0
    lax.fori_loop(0, ZR, zbody, 0)

    def pro_body(b, _):
        r0 = row0 + b * R
        sl = pl.ds(r0, R)
        pltpu.sync_copy(y2.at[c].at[sl], wbuf)
        pltpu.sync_copy(maskf.at[sl], mbuf)
        for g in range(R // L):
            mv = mbuf[pl.ds(g * L, L)]
            for l in range(L):
                bexp[g * L + l] = jnp.full((L,), mv[l], jnp.float32)

        def mrow(i, _):
            mv = bexp[i]
            for cc in range(NCH):
                csl = pl.ds(cc * L, L)
                v = wbuf[i, csl] * mv
                wbuf[i, csl] = v
                rbuf[i, csl] = v * (1.0 - ALPHA)
            return 0
        lax.fori_loop(0, R, mrow, 0)
        pltpu.sync_copy(wbuf, bufA.at[c].at[sl])
        pltpu.sync_copy(rbuf, res_h.at[c].at[sl])
        for q in range(R // ZR):
            pltpu.sync_copy(zbuf, acc.at[pl.ds(r0 + q * ZR, ZR)])
        return 0
    with jax.named_scope("prologue"):
        lax.fori_loop(0, NRB, pro_body, 0)
        plsc.subcore_barrier()

    srcs = [bufA, bufB, bufA]
    dsts = [bufB, bufA, out2]
    for layer in range(N_LAYERS):
        cur = srcs[layer]
        nxt = dsts[layer]

        def stage_descs(k, slot):
            esl = pl.ds((b0 + k) * K, K)
            return (
                pltpu.make_async_copy(srcs_h.at[esl], src2.at[slot], tsems[slot]),
                pltpu.make_async_copy(dsts_h.at[esl], dst2.at[slot], tsems[slot]),
                pltpu.make_async_copy(ws_h.at[esl], w2.at[slot], tsems[slot]),
            )

        def gather_desc(slot):
            return pltpu.make_async_copy(
                cur.at[c].at[src2.at[slot]], rowbuf2.at[slot], gsems[slot])

        def scat_issue(slot):
            pltpu.async_copy(
                rowbuf2.at[slot], acc.at[dst2.at[slot]], ssems[slot],
                add=True)

        def scat_wait(slot):
            pltpu.make_async_copy(
                rowbuf2.at[slot], acc.at[dst2.at[slot]], ssems[slot]).wait()

        def issue_stage(k, slot):
            for d in stage_descs(k, slot):
                d.start()

        def wait_stage(k, slot):
            for d in stage_descs(k, slot):
                d.wait()

        def mask_scale(slot):
            dref = dst2.at[slot]
            wref = w2.at[slot]
            for g in range(K // L):
                gsl = pl.ds(g * L, L)
                dv = dref[gsl]
                wv = wref[gsl]
                owned = (dv >= row0) & (dv < row0 + RPT)
                dref[gsl] = jnp.where(owned, dv, row0)
                wm = jnp.where(owned, wv, 0.0)
                for l in range(L):
                    bexp[g * L + l] = jnp.full((L,), wm[l], jnp.float32)
            rb = rowbuf2.at[slot]

            def srow(i, _):
                wv = bexp[i]
                for cc in range(NCH):
                    csl = pl.ds(cc * L, L)
                    rb[i, csl] = rb[i, csl] * wv
                return 0
            lax.fori_loop(0, K, srow, 0)

        @pl.when(nb > 0)
        def _():
            issue_stage(0, 0)
            wait_stage(0, 0)
            gather_desc(0).start()

        @pl.when(nb > 1)
        def _():
            issue_stage(1, 1)

        def pair_body(p, _):
            for q in (0, 1):
                k = 2 * p + q
                slot = q
                other = 1 - q

                @pl.when(k < nb)
                def _():
                    gather_desc(slot).wait()
                    mask_scale(slot)

                    @pl.when(k + 1 < nb)
                    def _():
                        wait_stage(k + 1, other)

                        @pl.when(k >= 1)
                        def _():
                            scat_wait(other)
                        gather_desc(other).start()

                    @pl.when(k + 2 < nb)
                    def _():
                        issue_stage(k + 2, slot)
                    scat_issue(slot)
            return 0
        with jax.named_scope(f"scatter{layer}"):
            lax.fori_loop(0, (nb + 1) // 2, pair_body, 0)

        @pl.when(nb >= 2)
        def _():
            scat_wait(0)
            scat_wait(1)

        @pl.when(nb == 1)
        def _():
            scat_wait(0)
        plsc.subcore_barrier()

        def wb_batch(b, _):
            r0 = row0 + b * R
            sl = pl.ds(r0, R)
            pltpu.sync_copy(acc.at[sl], wbuf)
            pltpu.sync_copy(res_h.at[c].at[sl], rbuf)

            def wb_row(i, _):
                for cc in range(NCH):
                    csl = pl.ds(cc * L, L)
                    v = wbuf[i, csl] * ALPHA + rbuf[i, csl]
                    wbuf[i, csl] = jnp.clip(v, 0.0, 1.0)
                return 0
            lax.fori_loop(0, R, wb_row, 0)
            pltpu.sync_copy(wbuf, nxt.at[c].at[sl])
            if layer < N_LAYERS - 1:
                for q in range(R // ZR):
                    pltpu.sync_copy(zbuf, acc.at[pl.ds(r0 + q * ZR, ZR)])
            return 0
        with jax.named_scope(f"writeback{layer}"):
            lax.fori_loop(0, NRB, wb_batch, 0)
            plsc.subcore_barrier()


@jax.jit
def _lpa_call(y2, maskf, src_s, dst_s, w_s, bnds):
    mesh = plsc.VectorSubcoreMesh(
        core_axis_name="c", subcore_axis_name="s",
        num_cores=NC, num_subcores=NS)
    f32 = jnp.float32
    i32 = jnp.int32
    out_types = (
        jax.ShapeDtypeStruct((NC, NP, CH), f32),
        jax.ShapeDtypeStruct((NC, NP, CH), f32),
        jax.ShapeDtypeStruct((NC, NP, CH), f32),
        jax.ShapeDtypeStruct((NC, NP, CH), f32),
    )
    scratch = [
        pltpu.VMEM_SHARED((NP, CH), f32),
        pltpu.VMEM((2, K), i32),
        pltpu.VMEM((2, K), i32),
        pltpu.VMEM((2, K), f32),
        pltpu.VMEM((L,), i32),
        pltpu.VMEM((2, K, CH), f32),
        pltpu.VMEM((R, CH), f32),
        pltpu.VMEM((R, CH), f32),
        pltpu.VMEM((ZR, CH), f32),
        pltpu.VMEM((R,), f32),
        pltpu.VMEM((K, L), f32),
        pltpu.SemaphoreType.DMA,
        pltpu.SemaphoreType.DMA,
        pltpu.SemaphoreType.DMA,
        pltpu.SemaphoreType.DMA,
        pltpu.SemaphoreType.DMA,
        pltpu.SemaphoreType.DMA,
    ]
    fn = pl.kernel(_lpa_body, out_type=out_types, mesh=mesh,
                   scratch_types=scratch)
    return fn(y2, maskf, src_s, dst_s, w_s, bnds)[0]


def kernel(y, edge_index, edge_weight, mask):
    y2 = y.reshape(N, NC, CH).transpose(1, 0, 2)
    y2 = jnp.pad(y2, ((0, 0), (0, NP - N), (0, 0)))
    maskf = jnp.pad(mask.astype(jnp.float32), (0, NP - N))
    dsti = edge_index[0].astype(jnp.int32)
    srci = edge_index[1].astype(jnp.int32)
    wf = edge_weight.astype(jnp.float32)
    dst_s, src_s, w_s = lax.sort((dsti, srci, wf), num_keys=1)
    bounds = jnp.searchsorted(
        dst_s, jnp.arange(0, NP + 1, RPT, dtype=jnp.int32)).astype(jnp.int32)
    bnds = jnp.stack([bounds[:NS], bounds[1:]], axis=1)
    bnds = jnp.pad(bnds, ((0, 0), (0, L - 2)))
    out2 = _lpa_call(y2, maskf, src_s, dst_s, w_s, bnds)
    return out2.transpose(1, 0, 2).reshape(NP, C)[:N]


# speedup vs baseline: 1.0277x; 1.0277x over previous
import jax
import jax.numpy as jnp
from jax import lax
from jax.experimental import pallas as pl
from jax.experimental.pallas import tpu as pltpu
from jax.experimental.pallas import tpu_sc as plsc

N = 10000
E = 160000
C = 256
N_LAYERS = 3
ALPHA = 0.9

NC = 2
NS = 16
L = 16
CH = C // NC
NP = 10240
K = 64
RPT = NP // NS
R = 32
NRB = RPT // R
ZR = 16
NCH = CH // L


def _lpa_body(y2, maskf, srcs_h, dsts_h, ws_h, bnds_h,
              out2, bufA, bufB, res_h,
              acc, src2, dst2, w2, bnds_v,
              rowbuf2, wbuf, rbuf, zbuf, mbuf, bexp,
              gsem0, gsem1, ssem0, ssem1, tsem0, tsem1):
    c = lax.axis_index("c")
    s = lax.axis_index("s")
    row0 = s * RPT
    gsems = (gsem0, gsem1)
    ssems = (ssem0, ssem1)
    tsems = (tsem0, tsem1)

    pltpu.sync_copy(bnds_h.at[s], bnds_v)
    bv = bnds_v[...]
    start = bv[0]
    end = bv[1]
    b0 = start // K
    b1 = jnp.where(end > start, (end - 1) // K + 1, b0)
    nb = b1 - b0

    def zbody(i, _):
        for cc in range(NCH):
            zbuf[i, pl.ds(cc * L, L)] = jnp.zeros((L,), jnp.float32)
        return 0
    lax.fori_loop(0, ZR, zbody, 0)

    def pro_body(b, _):
        r0 = row0 + b * R
        sl = pl.ds(r0, R)
        pltpu.sync_copy(y2.at[c].at[sl], wbuf)
        pltpu.sync_copy(maskf.at[sl], mbuf)
        for g in range(R // L):
            mv = mbuf[pl.ds(g * L, L)]
            for l in range(L):
                bexp[g * L + l] = jnp.full((L,), mv[l], jnp.float32)

        def mrow(i, _):
            mv = bexp[i]
            for cc in range(NCH):
                csl = pl.ds(cc * L, L)
                v = wbuf[i, csl] * mv
                wbuf[i, csl] = v
                rbuf[i, csl] = v * (1.0 - ALPHA)
            return 0
        lax.fori_loop(0, R, mrow, 0)
        pltpu.sync_copy(wbuf, bufA.at[c].at[sl])
        pltpu.sync_copy(rbuf, res_h.at[c].at[sl])
        for q in range(R // ZR):
            pltpu.sync_copy(zbuf, acc.at[pl.ds(r0 + q * ZR, ZR)])
        return 0
    lax.fori_loop(0, NRB, pro_body, 0)
    plsc.subcore_barrier()

    srcs = [bufA, bufB, bufA]
    dsts = [bufB, bufA, out2]
    for layer in range(N_LAYERS):
        cur = srcs[layer]
        nxt = dsts[layer]

        def stage_descs(k, slot):
            esl = pl.ds((b0 + k) * K, K)
            return (
                pltpu.make_async_copy(srcs_h.at[esl], src2.at[slot], tsems[slot]),
                pltpu.make_async_copy(dsts_h.at[esl], dst2.at[slot], tsems[slot]),
                pltpu.make_async_copy(ws_h.at[esl], w2.at[slot], tsems[slot]),
            )

        def gather_desc(slot):
            return pltpu.make_async_copy(
                cur.at[c].at[src2.at[slot]], rowbuf2.at[slot], gsems[slot])

        def scat_issue(slot):
            pltpu.async_copy(
                rowbuf2.at[slot], acc.at[dst2.at[slot]], ssems[slot],
                add=True)

        def scat_wait(slot):
            pltpu.make_async_copy(
                rowbuf2.at[slot], acc.at[dst2.at[slot]], ssems[slot]).wait()

        def issue_stage(k, slot):
            for d in stage_descs(k, slot):
                d.start()

        def wait_stage(k, slot):
            for d in stage_descs(k, slot):
                d.wait()

        def mask_scale(slot):
            dref = dst2.at[slot]
            wref = w2.at[slot]
            for g in range(K // L):
                gsl = pl.ds(g * L, L)
                dv = dref[gsl]
                wv = wref[gsl]
                owned = (dv >= row0) & (dv < row0 + RPT)
                dref[gsl] = jnp.where(owned, dv, row0)
                wm = jnp.where(owned, wv, 0.0)
                for l in range(L):
                    bexp[g * L + l] = jnp.full((L,), wm[l], jnp.float32)
            rb = rowbuf2.at[slot]

            def srow(i, _):
                wv = bexp[i]
                for cc in range(NCH):
                    csl = pl.ds(cc * L, L)
                    rb[i, csl] = rb[i, csl] * wv
                return 0
            lax.fori_loop(0, K, srow, 0)

        @pl.when(nb > 0)
        def _():
            issue_stage(0, 0)
            wait_stage(0, 0)
            gather_desc(0).start()

        @pl.when(nb > 1)
        def _():
            issue_stage(1, 1)

        def pair_body(p, _):
            for q in (0, 1):
                k = 2 * p + q
                slot = q
                other = 1 - q

                @pl.when(k < nb)
                def _():
                    gather_desc(slot).wait()
                    mask_scale(slot)

                    @pl.when(k + 1 < nb)
                    def _():
                        wait_stage(k + 1, other)

                        @pl.when(k >= 1)
                        def _():
                            scat_wait(other)
                        gather_desc(other).start()

                    @pl.when(k + 2 < nb)
                    def _():
                        issue_stage(k + 2, slot)
                    scat_issue(slot)
            return 0
        lax.fori_loop(0, (nb + 1) // 2, pair_body, 0)

        @pl.when(nb >= 2)
        def _():
            scat_wait(0)
            scat_wait(1)

        @pl.when(nb == 1)
        def _():
            scat_wait(0)
        plsc.subcore_barrier()

        def wb_batch(b, _):
            r0 = row0 + b * R
            sl = pl.ds(r0, R)
            pltpu.sync_copy(acc.at[sl], wbuf)
            pltpu.sync_copy(res_h.at[c].at[sl], rbuf)

            def wb_row(i, _):
                for cc in range(NCH):
                    csl = pl.ds(cc * L, L)
                    v = wbuf[i, csl] * ALPHA + rbuf[i, csl]
                    wbuf[i, csl] = jnp.clip(v, 0.0, 1.0)
                return 0
            lax.fori_loop(0, R, wb_row, 0)
            pltpu.sync_copy(wbuf, nxt.at[c].at[sl])
            if layer < N_LAYERS - 1:
                for q in range(R // ZR):
                    pltpu.sync_copy(zbuf, acc.at[pl.ds(r0 + q * ZR, ZR)])
            return 0
        lax.fori_loop(0, NRB, wb_batch, 0)
        plsc.subcore_barrier()


@jax.jit
def _lpa_call(y2, maskf, src_s, dst_s, w_s, bnds):
    mesh = plsc.VectorSubcoreMesh(
        core_axis_name="c", subcore_axis_name="s",
        num_cores=NC, num_subcores=NS)
    f32 = jnp.float32
    i32 = jnp.int32
    out_types = (
        jax.ShapeDtypeStruct((NC, NP, CH), f32),
        jax.ShapeDtypeStruct((NC, NP, CH), f32),
        jax.ShapeDtypeStruct((NC, NP, CH), f32),
        jax.ShapeDtypeStruct((NC, NP, CH), f32),
    )
    scratch = [
        pltpu.VMEM_SHARED((NP, CH), f32),
        pltpu.VMEM((2, K), i32),
        pltpu.VMEM((2, K), i32),
        pltpu.VMEM((2, K), f32),
        pltpu.VMEM((L,), i32),
        pltpu.VMEM((2, K, CH), f32),
        pltpu.VMEM((R, CH), f32),
        pltpu.VMEM((R, CH), f32),
        pltpu.VMEM((ZR, CH), f32),
        pltpu.VMEM((R,), f32),
        pltpu.VMEM((K, L), f32),
        pltpu.SemaphoreType.DMA,
        pltpu.SemaphoreType.DMA,
        pltpu.SemaphoreType.DMA,
        pltpu.SemaphoreType.DMA,
        pltpu.SemaphoreType.DMA,
        pltpu.SemaphoreType.DMA,
    ]
    fn = pl.kernel(_lpa_body, out_type=out_types, mesh=mesh,
                   scratch_types=scratch)
    return fn(y2, maskf, src_s, dst_s, w_s, bnds)[0]


def kernel(y, edge_index, edge_weight, mask):
    y2 = y.reshape(N, NC, CH).transpose(1, 0, 2)
    y2 = jnp.pad(y2, ((0, 0), (0, NP - N), (0, 0)))
    maskf = jnp.pad(mask.astype(jnp.float32), (0, NP - N))
    dsti = edge_index[0].astype(jnp.int32)
    srci = edge_index[1].astype(jnp.int32)
    wf = edge_weight.astype(jnp.float32)
    code = (dsti.astype(jnp.uint32) << 18) | jnp.arange(E, dtype=jnp.uint32)
    code = lax.sort(code)
    dst_s = (code >> 18).astype(jnp.int32)
    perm = (code & jnp.uint32((1 << 18) - 1)).astype(jnp.int32)
    src_s = jnp.take(srci, perm)
    w_s = jnp.take(wf, perm)
    bounds = jnp.searchsorted(
        dst_s, jnp.arange(0, NP + 1, RPT, dtype=jnp.int32)).astype(jnp.int32)
    bnds = jnp.stack([bounds[:NS], bounds[1:]], axis=1)
    bnds = jnp.pad(bnds, ((0, 0), (0, L - 2)))
    out2 = _lpa_call(y2, maskf, src_s, dst_s, w_s, bnds)
    return out2.transpose(1, 0, 2).reshape(NP, C)[:N]
